# baseline (speedup 1.0000x reference)
import functools
import jax
import jax.numpy as jnp
from jax import lax
from jax.experimental import pallas as pl
from jax.experimental.pallas import tpu as pltpu
from jax.experimental.pallas import tpu_sc as plsc

N = 10000
E = 320000
D = 64
DIN = 128
DOUT = 128
G = 64

NC = 2
NS = 16
NW = NC * NS

NP = 10240
ROWS_PER_TILE = NP // NS
CH = 128
TCH = 2 * (-(-E // (NW * CH * 2)))
EP = NW * CH * TCH

_mesh = plsc.VectorSubcoreMesh(core_axis_name="c", subcore_axis_name="s")


ZR = 128


def _zero_fill_zbuf(zbuf):
    for i in range(ZR):
        for k in range(D // 16):
            zbuf[i, pl.ds(k * 16, 16)] = jnp.zeros((16,), jnp.float32)


@functools.partial(
    pl.kernel,
    out_type=jax.ShapeDtypeStruct((NC, NP, D), jnp.float32),
    mesh=_mesh,
    scratch_types=[
        pltpu.VMEM((TCH, 2, CH), jnp.int32),
        pltpu.VMEM((4, CH, D), jnp.float32),
        pltpu.VMEM((ZR, D), jnp.float32),
        pltpu.VMEM_SHARED((NP, D), jnp.float32),
        pltpu.SemaphoreType.DMA,
        pltpu.SemaphoreType.DMA,
        pltpu.SemaphoreType.DMA,
        pltpu.SemaphoreType.DMA,
        pltpu.SemaphoreType.DMA,
        pltpu.SemaphoreType.DMA,
        pltpu.SemaphoreType.DMA,
        pltpu.SemaphoreType.DMA,
    ],
    compiler_params=pltpu.CompilerParams(use_tc_tiling_on_sc=False),
)
def _sc_aggregate(u_hbm, edges_hbm, out_hbm,
                  eidx, rows, zbuf, shared,
                  g0, g1, g2, g3, s0, s1, s2, s3):
    c = lax.axis_index("c")
    s = lax.axis_index("s")
    tile = c * NS + s
    gsems = (g0, g1, g2, g3)
    ssems = (s0, s1, s2, s3)

    pltpu.async_copy(edges_hbm.at[:, tile, :, :], eidx, g0)

    _zero_fill_zbuf(zbuf)
    base = s * ROWS_PER_TILE
    for j in range(ROWS_PER_TILE // ZR):
        pltpu.sync_copy(zbuf, shared.at[pl.ds(base + j * ZR, ZR), :])
    pltpu.make_async_copy(edges_hbm.at[:, tile, :, :], eidx, g0).wait()
    plsc.subcore_barrier()

    def _gather(i, j):
        pltpu.async_copy(u_hbm.at[eidx.at[i, 0]], rows.at[j], gsems[j])

    def _wait_gather(j):
        pltpu.make_async_copy(u_hbm.at[eidx.at[0, 0]], rows.at[j],
                              gsems[j]).wait()

    def _scatter(i, j):
        pltpu.async_copy(rows.at[j], shared.at[eidx.at[i, 1]], ssems[j],
                         add=True)

    def _wait_scatter(j):
        pltpu.make_async_copy(rows.at[j], shared.at[eidx.at[0, 1]],
                              ssems[j]).wait()

    for j in range(3):
        _gather(j, j)

    def group_body(k, carry):
        i = 4 * k
        _gather(jnp.minimum(i + 3, TCH - 1), 3)
        for j in range(4):
            _wait_gather(j)
            _scatter(i + j, j)
        for j in range(3):
            _wait_scatter(j)
            _gather(jnp.minimum(i + 4 + j, TCH - 1), j)
        _wait_scatter(3)
        return carry

    lax.fori_loop(0, TCH // 4, group_body, 0)
    for j in range(3):
        _wait_gather(j)
    plsc.subcore_barrier()

    pltpu.sync_copy(shared.at[pl.ds(base, ROWS_PER_TILE), :],
                    out_hbm.at[c, pl.ds(base, ROWS_PER_TILE), :])


@functools.partial(
    pl.kernel,
    out_type=jax.ShapeDtypeStruct((NC, NP), jnp.float32),
    mesh=_mesh,
    scratch_types=[
        pltpu.VMEM((TCH, 2, CH), jnp.int32),
        pltpu.VMEM((CH,), jnp.float32),
        pltpu.VMEM((ZR,), jnp.float32),
        pltpu.VMEM_SHARED((NP,), jnp.float32),
        pltpu.SemaphoreType.DMA,
    ],
    compiler_params=pltpu.CompilerParams(use_tc_tiling_on_sc=False),
)
def _sc_degree(edges_hbm, out_hbm, eidx, ones_v, zb, shared, sem):
    c = lax.axis_index("c")
    s = lax.axis_index("s")
    tile = c * NS + s

    pltpu.sync_copy(edges_hbm.at[:, tile, :, :], eidx)

    for k in range(ZR // 16):
        zb[pl.ds(k * 16, 16)] = jnp.zeros((16,), jnp.float32)
    for k in range(CH // 16):
        ones_v[pl.ds(k * 16, 16)] = jnp.ones((16,), jnp.float32)

    base = s * ROWS_PER_TILE
    for j in range(ROWS_PER_TILE // ZR):
        pltpu.sync_copy(zb, shared.at[pl.ds(base + j * ZR, ZR)])
    plsc.subcore_barrier()

    def chunk_body(i, carry):
        pltpu.async_copy(ones_v, shared.at[eidx.at[i, 1]], sem, add=True)
        return carry

    lax.fori_loop(0, TCH, chunk_body, 0)

    def drain_body(i, carry):
        pltpu.make_async_copy(ones_v, shared.at[eidx.at[0, 1]], sem).wait()
        return carry

    lax.fori_loop(0, TCH, drain_body, 0)
    plsc.subcore_barrier()

    pltpu.sync_copy(shared.at[pl.ds(base, ROWS_PER_TILE)],
                    out_hbm.at[c, pl.ds(base, ROWS_PER_TILE)])


MP = N // 2
BP = 200
NB = MP // BP
L2 = 2 * D

_spec_sp = pl.BlockSpec((NC, BP, L2), lambda i: (0, i, 0))
_spec_p = pl.BlockSpec((BP, L2), lambda i: (i, 0))
_spec_p1 = pl.BlockSpec((BP, 1), lambda i: (i, 0))
_spec_b2 = pl.BlockSpec((1, L2), lambda i: (0, 0))
_spec_w2 = pl.BlockSpec((L2, L2), lambda i: (0, 0))


def _lane_scale(de, do):
    lane = lax.broadcasted_iota(jnp.int32, (BP, L2), 1)
    return jnp.where(lane < D, de, do)


def _mm1_body(x_ref, w_ref, o_ref):
    o_ref[...] = jnp.dot(x_ref[...], w_ref[...],
                         preferred_element_type=jnp.float32)


def _u0_body(xw_ref, dpair_ref, u_ref, dinve_ref, dinvo_ref):
    dinv_e = lax.rsqrt(dpair_ref[:, 0:1] + 1.0)
    dinv_o = lax.rsqrt(dpair_ref[:, 1:2] + 1.0)
    dinve_ref[...] = dinv_e
    dinvo_ref[...] = dinv_o
    u_ref[...] = xw_ref[...] * _lane_scale(dinv_e, dinv_o)


def _layer_body(s_ref, u_ref, de_ref, do_ref, b_ref, w_ref, o_ref):
    scale = _lane_scale(de_ref[...], do_ref[...])
    t = s_ref[0] + s_ref[1] + u_ref[...]
    h = jnp.maximum(t * scale + b_ref[...], 0.0)
    o_ref[...] = jnp.dot(h, w_ref[...],
                         preferred_element_type=jnp.float32) * scale


def _layer4_body(s_ref, u_ref, de_ref, do_ref, b_ref, o_ref):
    scale = _lane_scale(de_ref[...], do_ref[...])
    t = s_ref[0] + s_ref[1] + u_ref[...]
    h = jnp.maximum(t * scale + b_ref[...], 0.0)
    o_ref[...] = h * scale


def _pool_body(s_ref, u_ref, de_ref, do_ref, bpair_ref, w_ref, b_ref,
               o_ref, sums_acc, cnt_acc):
    i = pl.program_id(0)

    @pl.when(i == 0)
    def _init():
        sums_acc[...] = jnp.zeros((G, D), jnp.float32)
        cnt_acc[...] = jnp.zeros((G, 128), jnp.float32)

    z = (s_ref[0] + s_ref[1] + u_ref[...]) * _lane_scale(de_ref[...],
                                                         do_ref[...])
    gids = lax.broadcasted_iota(jnp.int32, (1, G), 1)
    oh_e = (bpair_ref[:, 0:1] == gids).astype(jnp.float32)
    oh_o = (bpair_ref[:, 1:2] == gids).astype(jnp.float32)
    sums_acc[...] += (
        lax.dot_general(oh_e, z[:, :D], (((0,), (0,)), ((), ())),
                        preferred_element_type=jnp.float32)
        + lax.dot_general(oh_o, z[:, D:], (((0,), (0,)), ((), ())),
                          preferred_element_type=jnp.float32))
    cnt_acc[...] += jnp.sum(oh_e + oh_o, axis=0)[:, None]

    @pl.when(i == NB - 1)
    def _final():
        mean = sums_acc[...] / jnp.maximum(cnt_acc[:, :1], 1.0)
        o_ref[...] = jnp.dot(mean, w_ref[...],
                             preferred_element_type=jnp.float32) + b_ref[...]


def _mm1(x2, w1d):
    return pl.pallas_call(
        _mm1_body,
        grid=(NB,),
        in_specs=[pl.BlockSpec((BP, 2 * DIN), lambda i: (i, 0)),
                  pl.BlockSpec((2 * DIN, L2), lambda i: (0, 0))],
        out_specs=_spec_p,
        out_shape=jax.ShapeDtypeStruct((MP, L2), jnp.float32),
    )(x2, w1d)


_spec_p2 = pl.BlockSpec((BP, 2), lambda i: (i, 0))


def _u0(xwp, deg_pair):
    return pl.pallas_call(
        _u0_body,
        grid=(NB,),
        in_specs=[_spec_p, _spec_p2],
        out_specs=(_spec_p, _spec_p1, _spec_p1),
        out_shape=(jax.ShapeDtypeStruct((MP, L2), jnp.float32),
                   jax.ShapeDtypeStruct((MP, 1), jnp.float32),
                   jax.ShapeDtypeStruct((MP, 1), jnp.float32)),
    )(xwp, deg_pair)


def _layer(sp, up, de, do, b2, w2):
    return pl.pallas_call(
        _layer_body,
        grid=(NB,),
        in_specs=[_spec_sp, _spec_p, _spec_p1, _spec_p1, _spec_b2, _spec_w2],
        out_specs=_spec_p,
        out_shape=jax.ShapeDtypeStruct((MP, L2), jnp.float32),
    )(sp, up, de, do, b2, w2)


def _layer4(sp, up, de, do, b2):
    return pl.pallas_call(
        _layer4_body,
        grid=(NB,),
        in_specs=[_spec_sp, _spec_p, _spec_p1, _spec_p1, _spec_b2],
        out_specs=_spec_p,
        out_shape=jax.ShapeDtypeStruct((MP, L2), jnp.float32),
    )(sp, up, de, do, b2)


def _pool(sp, up, de, do, batch_pair, w, b):
    return pl.pallas_call(
        _pool_body,
        grid=(NB,),
        in_specs=[_spec_sp, _spec_p, _spec_p1, _spec_p1, _spec_p2,
                  pl.BlockSpec((D, DOUT), lambda i: (0, 0)),
                  pl.BlockSpec((1, DOUT), lambda i: (0, 0))],
        out_specs=pl.BlockSpec((G, DOUT), lambda i: (0, 0)),
        out_shape=jax.ShapeDtypeStruct((G, DOUT), jnp.float32),
        scratch_shapes=[pltpu.VMEM((G, D), jnp.float32),
                        pltpu.VMEM((G, 128), jnp.float32)],
    )(sp, up, de, do, batch_pair, w, b)


def _blockdiag(w):
    k = w.shape[0]
    z = jnp.zeros((k, D), jnp.float32)
    return jnp.concatenate(
        [jnp.concatenate([w, z], axis=1),
         jnp.concatenate([z, w], axis=1)], axis=0)


def kernel(x, edge_index, batch, W1, b1, W2, b2, W3, b3, W4, b4):
    pad = EP - E
    pad_src = (jnp.arange(pad, dtype=jnp.int32) * 61) % N
    pad_dst = N + (jnp.arange(pad, dtype=jnp.int32) % (NP - N))
    ei3 = edge_index.reshape(2, E // CH, CH).transpose(1, 0, 2)
    padc = jnp.stack([pad_src.reshape(pad // CH, CH),
                      pad_dst.reshape(pad // CH, CH)], axis=1)
    edges = jnp.concatenate([ei3, padc], axis=0).reshape(TCH, NW, 2, CH)

    x2 = x.reshape(MP, 2 * DIN)
    w1d = _blockdiag(W1)
    w2d = _blockdiag(W2)
    w3d = _blockdiag(W3)
    b1d = jnp.concatenate([b1, b1])[None, :]
    b2d = jnp.concatenate([b2, b2])[None, :]
    b3d = jnp.concatenate([b3, b3])[None, :]

    deg2 = _sc_degree(edges)
    deg_pair = (deg2[0, :N] + deg2[1, :N]).reshape(MP, 2)
    batch_pair = batch.reshape(MP, 2)

    xwp = _mm1(x2, w1d)
    u0p, de, do = _u0(xwp, deg_pair)

    s1 = _sc_aggregate(u0p.reshape(N, D), edges)
    u1p = _layer(s1.reshape(NC, NP // 2, L2), u0p, de, do, b1d, w2d)
    s2 = _sc_aggregate(u1p.reshape(N, D), edges)
    u2p = _layer(s2.reshape(NC, NP // 2, L2), u1p, de, do, b2d, w3d)
    s3 = _sc_aggregate(u2p.reshape(N, D), edges)
    u3p = _layer4(s3.reshape(NC, NP // 2, L2), u2p, de, do, b3d)
    s4 = _sc_aggregate(u3p.reshape(N, D), edges)
    out = _pool(s4.reshape(NC, NP // 2, L2), u3p, de, do,
                batch_pair, W4, b4[None, :])
    return out


# speedup vs baseline: 1.1881x; 1.1881x over previous
import functools
import jax
import jax.numpy as jnp
from jax import lax
from jax.experimental import pallas as pl
from jax.experimental.pallas import tpu as pltpu
from jax.experimental.pallas import tpu_sc as plsc

N = 10000
E = 320000
D = 64
DIN = 128
DOUT = 128
G = 64

NC = 2
NS = 16
NW = NC * NS

NP = 10240
ROWS_PER_TILE = NP // NS
CH = 128
TCH = 2 * (-(-E // (NW * CH * 2)))
EP = NW * CH * TCH

_mesh = plsc.VectorSubcoreMesh(core_axis_name="c", subcore_axis_name="s")


ZR = 128


def _zero_fill_zbuf(zbuf):
    for i in range(ZR):
        for k in range(D // 16):
            zbuf[i, pl.ds(k * 16, 16)] = jnp.zeros((16,), jnp.float32)


@functools.partial(
    pl.kernel,
    out_type=jax.ShapeDtypeStruct((NC, NP, D), jnp.float32),
    mesh=_mesh,
    scratch_types=[
        pltpu.VMEM((TCH, 2, CH), jnp.int32),
        pltpu.VMEM((4, CH, D), jnp.float32),
        pltpu.VMEM((ZR, D), jnp.float32),
        pltpu.VMEM_SHARED((NP, D), jnp.float32),
        pltpu.SemaphoreType.DMA,
        pltpu.SemaphoreType.DMA,
        pltpu.SemaphoreType.DMA,
        pltpu.SemaphoreType.DMA,
        pltpu.SemaphoreType.DMA,
        pltpu.SemaphoreType.DMA,
        pltpu.SemaphoreType.DMA,
        pltpu.SemaphoreType.DMA,
    ],
    compiler_params=pltpu.CompilerParams(use_tc_tiling_on_sc=False),
)
def _sc_aggregate(u_hbm, edges_hbm, out_hbm,
                  eidx, rows, zbuf, shared,
                  g0, g1, g2, g3, s0, s1, s2, s3):
    c = lax.axis_index("c")
    s = lax.axis_index("s")
    tile = c * NS + s
    gsems = (g0, g1, g2, g3)
    ssems = (s0, s1, s2, s3)

    pltpu.async_copy(edges_hbm.at[:, tile, :, :], eidx, g0)

    _zero_fill_zbuf(zbuf)
    base = s * ROWS_PER_TILE
    for j in range(ROWS_PER_TILE // ZR):
        pltpu.sync_copy(zbuf, shared.at[pl.ds(base + j * ZR, ZR), :])
    pltpu.make_async_copy(edges_hbm.at[:, tile, :, :], eidx, g0).wait()
    plsc.subcore_barrier()

    def _gather(i, j):
        pltpu.async_copy(u_hbm.at[eidx.at[i, 0]], rows.at[j], gsems[j])

    def _wait_gather(j):
        pltpu.make_async_copy(u_hbm.at[eidx.at[0, 0]], rows.at[j],
                              gsems[j]).wait()

    def _scatter(i, j):
        pltpu.async_copy(rows.at[j], shared.at[eidx.at[i, 1]], ssems[j],
                         add=True)

    def _wait_scatter(j):
        pltpu.make_async_copy(rows.at[j], shared.at[eidx.at[0, 1]],
                              ssems[j]).wait()

    for j in range(3):
        _gather(j, j)

    def group_body(k, carry):
        i = 4 * k
        _gather(jnp.minimum(i + 3, TCH - 1), 3)
        for j in range(4):
            _wait_gather(j)
            _scatter(i + j, j)
        for j in range(3):
            _wait_scatter(j)
            _gather(jnp.minimum(i + 4 + j, TCH - 1), j)
        _wait_scatter(3)
        return carry

    lax.fori_loop(0, TCH // 4, group_body, 0)
    for j in range(3):
        _wait_gather(j)
    plsc.subcore_barrier()

    pltpu.sync_copy(shared.at[pl.ds(base, ROWS_PER_TILE), :],
                    out_hbm.at[c, pl.ds(base, ROWS_PER_TILE), :])


@functools.partial(
    pl.kernel,
    out_type=jax.ShapeDtypeStruct((NC, NP), jnp.float32),
    mesh=_mesh,
    scratch_types=[
        pltpu.VMEM((TCH, 2, CH), jnp.int32),
        pltpu.VMEM((CH,), jnp.float32),
        pltpu.VMEM((ZR,), jnp.float32),
        pltpu.VMEM_SHARED((NP,), jnp.float32),
        pltpu.SemaphoreType.DMA,
    ],
    compiler_params=pltpu.CompilerParams(use_tc_tiling_on_sc=False),
)
def _sc_degree(edges_hbm, out_hbm, eidx, ones_v, zb, shared, sem):
    c = lax.axis_index("c")
    s = lax.axis_index("s")
    tile = c * NS + s

    pltpu.sync_copy(edges_hbm.at[:, tile, :, :], eidx)

    for k in range(ZR // 16):
        zb[pl.ds(k * 16, 16)] = jnp.zeros((16,), jnp.float32)
    for k in range(CH // 16):
        ones_v[pl.ds(k * 16, 16)] = jnp.ones((16,), jnp.float32)

    base = s * ROWS_PER_TILE
    for j in range(ROWS_PER_TILE // ZR):
        pltpu.sync_copy(zb, shared.at[pl.ds(base + j * ZR, ZR)])
    plsc.subcore_barrier()

    def chunk_body(i, carry):
        pltpu.async_copy(ones_v, shared.at[eidx.at[i, 1]], sem, add=True)
        return carry

    lax.fori_loop(0, TCH, chunk_body, 0)

    def drain_body(i, carry):
        pltpu.make_async_copy(ones_v, shared.at[eidx.at[0, 1]], sem).wait()
        return carry

    lax.fori_loop(0, TCH, drain_body, 0)
    plsc.subcore_barrier()

    pltpu.sync_copy(shared.at[pl.ds(base, ROWS_PER_TILE)],
                    out_hbm.at[c, pl.ds(base, ROWS_PER_TILE)])


MP = N // 2
BP = 1000
NB = MP // BP
L2 = 2 * D

_spec_sp = pl.BlockSpec((NC, BP, L2), lambda i: (0, i, 0))
_spec_p = pl.BlockSpec((BP, L2), lambda i: (i, 0))
_spec_p1 = pl.BlockSpec((BP, 1), lambda i: (i, 0))
_spec_b2 = pl.BlockSpec((1, L2), lambda i: (0, 0))
_spec_w2 = pl.BlockSpec((L2, L2), lambda i: (0, 0))


def _lane_scale(de, do):
    lane = lax.broadcasted_iota(jnp.int32, (BP, L2), 1)
    return jnp.where(lane < D, de, do)


def _mm1_body(x_ref, w_ref, o_ref):
    o_ref[...] = jnp.dot(x_ref[...], w_ref[...],
                         preferred_element_type=jnp.float32)


def _u0_body(xw_ref, dpair_ref, u_ref, dinve_ref, dinvo_ref):
    dinv_e = lax.rsqrt(dpair_ref[:, 0:1] + 1.0)
    dinv_o = lax.rsqrt(dpair_ref[:, 1:2] + 1.0)
    dinve_ref[...] = dinv_e
    dinvo_ref[...] = dinv_o
    u_ref[...] = xw_ref[...] * _lane_scale(dinv_e, dinv_o)


def _layer_body(s_ref, u_ref, de_ref, do_ref, b_ref, w_ref, o_ref):
    scale = _lane_scale(de_ref[...], do_ref[...])
    t = s_ref[0] + s_ref[1] + u_ref[...]
    h = jnp.maximum(t * scale + b_ref[...], 0.0)
    o_ref[...] = jnp.dot(h, w_ref[...],
                         preferred_element_type=jnp.float32) * scale


def _layer4_body(s_ref, u_ref, de_ref, do_ref, b_ref, o_ref):
    scale = _lane_scale(de_ref[...], do_ref[...])
    t = s_ref[0] + s_ref[1] + u_ref[...]
    h = jnp.maximum(t * scale + b_ref[...], 0.0)
    o_ref[...] = h * scale


def _pool_body(s_ref, u_ref, de_ref, do_ref, bpair_ref, w_ref, b_ref,
               o_ref, sums_acc, cnt_acc):
    i = pl.program_id(0)

    @pl.when(i == 0)
    def _init():
        sums_acc[...] = jnp.zeros((G, D), jnp.float32)
        cnt_acc[...] = jnp.zeros((G, 128), jnp.float32)

    z = (s_ref[0] + s_ref[1] + u_ref[...]) * _lane_scale(de_ref[...],
                                                         do_ref[...])
    gids = lax.broadcasted_iota(jnp.int32, (1, G), 1)
    oh_e = (bpair_ref[:, 0:1] == gids).astype(jnp.float32)
    oh_o = (bpair_ref[:, 1:2] == gids).astype(jnp.float32)
    sums_acc[...] += (
        lax.dot_general(oh_e, z[:, :D], (((0,), (0,)), ((), ())),
                        preferred_element_type=jnp.float32)
        + lax.dot_general(oh_o, z[:, D:], (((0,), (0,)), ((), ())),
                          preferred_element_type=jnp.float32))
    cnt_acc[...] += jnp.sum(oh_e + oh_o, axis=0)[:, None]

    @pl.when(i == NB - 1)
    def _final():
        mean = sums_acc[...] / jnp.maximum(cnt_acc[:, :1], 1.0)
        o_ref[...] = jnp.dot(mean, w_ref[...],
                             preferred_element_type=jnp.float32) + b_ref[...]


def _mm1(x2, w1d):
    return pl.pallas_call(
        _mm1_body,
        grid=(NB,),
        in_specs=[pl.BlockSpec((BP, 2 * DIN), lambda i: (i, 0)),
                  pl.BlockSpec((2 * DIN, L2), lambda i: (0, 0))],
        out_specs=_spec_p,
        out_shape=jax.ShapeDtypeStruct((MP, L2), jnp.float32),
    )(x2, w1d)


_spec_p2 = pl.BlockSpec((BP, 2), lambda i: (i, 0))


def _u0(xwp, deg_pair):
    return pl.pallas_call(
        _u0_body,
        grid=(NB,),
        in_specs=[_spec_p, _spec_p2],
        out_specs=(_spec_p, _spec_p1, _spec_p1),
        out_shape=(jax.ShapeDtypeStruct((MP, L2), jnp.float32),
                   jax.ShapeDtypeStruct((MP, 1), jnp.float32),
                   jax.ShapeDtypeStruct((MP, 1), jnp.float32)),
    )(xwp, deg_pair)


def _layer(sp, up, de, do, b2, w2):
    return pl.pallas_call(
        _layer_body,
        grid=(NB,),
        in_specs=[_spec_sp, _spec_p, _spec_p1, _spec_p1, _spec_b2, _spec_w2],
        out_specs=_spec_p,
        out_shape=jax.ShapeDtypeStruct((MP, L2), jnp.float32),
    )(sp, up, de, do, b2, w2)


def _layer4(sp, up, de, do, b2):
    return pl.pallas_call(
        _layer4_body,
        grid=(NB,),
        in_specs=[_spec_sp, _spec_p, _spec_p1, _spec_p1, _spec_b2],
        out_specs=_spec_p,
        out_shape=jax.ShapeDtypeStruct((MP, L2), jnp.float32),
    )(sp, up, de, do, b2)


def _pool(sp, up, de, do, batch_pair, w, b):
    return pl.pallas_call(
        _pool_body,
        grid=(NB,),
        in_specs=[_spec_sp, _spec_p, _spec_p1, _spec_p1, _spec_p2,
                  pl.BlockSpec((D, DOUT), lambda i: (0, 0)),
                  pl.BlockSpec((1, DOUT), lambda i: (0, 0))],
        out_specs=pl.BlockSpec((G, DOUT), lambda i: (0, 0)),
        out_shape=jax.ShapeDtypeStruct((G, DOUT), jnp.float32),
        scratch_shapes=[pltpu.VMEM((G, D), jnp.float32),
                        pltpu.VMEM((G, 128), jnp.float32)],
    )(sp, up, de, do, batch_pair, w, b)


def _blockdiag(w):
    k = w.shape[0]
    z = jnp.zeros((k, D), jnp.float32)
    return jnp.concatenate(
        [jnp.concatenate([w, z], axis=1),
         jnp.concatenate([z, w], axis=1)], axis=0)


def kernel(x, edge_index, batch, W1, b1, W2, b2, W3, b3, W4, b4):
    pad = EP - E
    pad_src = (jnp.arange(pad, dtype=jnp.int32) * 61) % N
    pad_dst = N + (jnp.arange(pad, dtype=jnp.int32) % (NP - N))
    ei3 = edge_index.reshape(2, E // CH, CH).transpose(1, 0, 2)
    padc = jnp.stack([pad_src.reshape(pad // CH, CH),
                      pad_dst.reshape(pad // CH, CH)], axis=1)
    edges = jnp.concatenate([ei3, padc], axis=0).reshape(TCH, NW, 2, CH)

    x2 = x.reshape(MP, 2 * DIN)
    w1d = _blockdiag(W1)
    w2d = _blockdiag(W2)
    w3d = _blockdiag(W3)
    b1d = jnp.concatenate([b1, b1])[None, :]
    b2d = jnp.concatenate([b2, b2])[None, :]
    b3d = jnp.concatenate([b3, b3])[None, :]

    deg2 = _sc_degree(edges)
    deg_pair = (deg2[0, :N] + deg2[1, :N]).reshape(MP, 2)
    batch_pair = batch.reshape(MP, 2)

    xwp = _mm1(x2, w1d)
    u0p, de, do = _u0(xwp, deg_pair)

    s1 = _sc_aggregate(u0p.reshape(N, D), edges)
    u1p = _layer(s1.reshape(NC, NP // 2, L2), u0p, de, do, b1d, w2d)
    s2 = _sc_aggregate(u1p.reshape(N, D), edges)
    u2p = _layer(s2.reshape(NC, NP // 2, L2), u1p, de, do, b2d, w3d)
    s3 = _sc_aggregate(u2p.reshape(N, D), edges)
    u3p = _layer4(s3.reshape(NC, NP // 2, L2), u2p, de, do, b3d)
    s4 = _sc_aggregate(u3p.reshape(N, D), edges)
    out = _pool(s4.reshape(NC, NP // 2, L2), u3p, de, do,
                batch_pair, W4, b4[None, :])
    return out
